# baseline (speedup 1.0000x reference)
import functools

import jax
import jax.numpy as jnp
from jax import lax
from jax.experimental import pallas as pl
from jax.experimental.pallas import tpu as pltpu
from jax.experimental.pallas import tpu_sc as plsc

N = 10000
D = 128
E = 320000
G = 64

NC = 2
NS = 16
NW = NC * NS

CH = 128
TOT_CHUNKS = NW * (-(-E // (NW * CH)))
EPAD = TOT_CHUNKS * CH
CPW_DEG = TOT_CHUNKS // NW
CPW = TOT_CHUNKS // NS

NHIST = 10240
NHALF = N // 2
NACC = 5120
RPT = NACC // NS
ZCH = 32
NBUF = 2
IGN = -1


def _sc_mesh():
    return plsc.VectorSubcoreMesh(
        core_axis_name="c", subcore_axis_name="s", num_cores=NC, num_subcores=NS
    )


def _sc_degree(dsts):
    @functools.partial(
        pl.kernel,
        out_type=jax.ShapeDtypeStruct((NW, NHIST), jnp.float32),
        mesh=_sc_mesh(),
        compiler_params=pltpu.CompilerParams(needs_layout_passes=False),
        scratch_types=[
            pltpu.VMEM((CPW_DEG, 1, CH), jnp.int32),
            pltpu.VMEM((NHIST,), jnp.float32),
        ],
    )
    def deg_kernel(dst_hbm, out_hbm, dbuf, hist):
        c = lax.axis_index("c")
        s = lax.axis_index("s")
        wid = c * NS + s
        base = wid * CPW_DEG
        pltpu.sync_copy(dst_hbm.at[pl.ds(base, CPW_DEG)], dbuf)
        zero16 = jnp.zeros((16,), jnp.float32)

        @pl.loop(0, NHIST // 16)
        def _zero(i):
            hist[pl.ds(i * 16, 16)] = zero16

        ones16 = jnp.ones((16,), jnp.float32)

        @pl.loop(0, CPW_DEG)
        def _edges(j):
            for v in range(CH // 16):
                idx = dbuf[j, 0, pl.ds(v * 16, 16)]
                plsc.addupdate_scatter(hist, [idx], ones16)

        pltpu.sync_copy(hist, out_hbm.at[wid])

    return deg_kernel(dsts)


def _sc_aggregate(m, srcs2, dsts2):
    @functools.partial(
        pl.kernel,
        out_type=jax.ShapeDtypeStruct((NC, NACC, D), jnp.float32),
        mesh=_sc_mesh(),
        scratch_types=[
            pltpu.VMEM((CPW, 1, CH), jnp.int32),
            pltpu.VMEM((CPW, 1, CH), jnp.int32),
            pltpu.VMEM((NBUF, CH, D), jnp.float32),
            pltpu.VMEM((ZCH, D), jnp.float32),
            pltpu.VMEM_SHARED((NACC, D), jnp.float32),
            pltpu.SemaphoreType.DMA,
            pltpu.SemaphoreType.DMA,
        ],
    )
    def agg_kernel(m_hbm, src_hbm, dst_hbm, out_hbm, sidx, didx, rows, zbuf,
                   acc, gsem, ssem):
        c = lax.axis_index("c")
        s = lax.axis_index("s")
        base = s * CPW
        pltpu.sync_copy(src_hbm.at[c, pl.ds(base, CPW)], sidx)
        pltpu.sync_copy(dst_hbm.at[c, pl.ds(base, CPW)], didx)
        zero16 = jnp.zeros((16,), jnp.float32)

        @pl.loop(0, ZCH)
        def _zrow(i):
            for v in range(D // 16):
                zbuf[i, pl.ds(v * 16, 16)] = zero16

        rbase = s * RPT

        @pl.loop(0, RPT // ZCH)
        def _zacc(i):
            pltpu.sync_copy(zbuf, acc.at[pl.ds(rbase + i * ZCH, ZCH)])

        plsc.subcore_barrier()

        def _gather(j, b):
            pltpu.async_copy(
                m_hbm.at[plsc.Indices(sidx.at[j, 0], ignored_value=IGN)],
                rows.at[b], gsem)

        def _scatter_desc(j, b):
            return pltpu.make_async_copy(
                rows.at[b],
                acc.at[plsc.Indices(didx.at[j, 0], ignored_value=IGN)],
                ssem)

        _gather(0, 0)

        @pl.loop(0, CPW)
        def _edges(j):
            b = lax.rem(j, NBUF)
            pltpu.make_async_copy(
                m_hbm.at[plsc.Indices(sidx.at[j, 0], ignored_value=IGN)],
                rows.at[b], gsem).wait()
            pltpu.async_copy(
                rows.at[b],
                acc.at[plsc.Indices(didx.at[j, 0], ignored_value=IGN)],
                ssem, add=True)

            @pl.when(j + 1 < CPW)
            def _refill():
                @pl.when(j >= 1)
                def _drain():
                    _scatter_desc(j - 1, 1 - b).wait()

                _gather(j + 1, 1 - b)

        _scatter_desc(CPW - 2, (CPW - 2) % NBUF).wait()
        _scatter_desc(CPW - 1, (CPW - 1) % NBUF).wait()

        plsc.subcore_barrier()
        pltpu.sync_copy(acc.at[pl.ds(rbase, RPT)], out_hbm.at[c, pl.ds(rbase, RPT)])

    return agg_kernel(m, srcs2, dsts2)


def _tc_first(x, W1, deg_t):
    def body(x_ref, w_ref, deg_ref, m_ref, dinv_ref):
        deg = jnp.sum(deg_ref[...], axis=1, keepdims=True) + 1.0
        dinv = lax.rsqrt(jnp.maximum(deg, 1.0))
        dinv_ref[...] = dinv
        h = jnp.dot(x_ref[...], w_ref[...], preferred_element_type=jnp.float32)
        m_ref[...] = h * dinv

    return pl.pallas_call(
        body,
        out_shape=[
            jax.ShapeDtypeStruct((N, D), jnp.float32),
            jax.ShapeDtypeStruct((N, 1), jnp.float32),
        ],
    )(x, W1, deg_t)


def _combine(acc_ref, m_ref):
    agg = jnp.concatenate(
        [acc_ref[0, pl.ds(0, NHALF)], acc_ref[1, pl.ds(0, NHALF)]], axis=0)
    return agg + m_ref[...]


def _tc_mid(acc, m, dinv, b, a, w_next):
    def body(acc_ref, m_ref, dinv_ref, b_ref, a_ref, w_ref, out_ref):
        dinv = dinv_ref[...]
        t = _combine(acc_ref, m_ref) * dinv + b_ref[...]
        z = jnp.where(t >= 0, t, a_ref[0, 0] * t)
        h = jnp.dot(z, w_ref[...], preferred_element_type=jnp.float32)
        out_ref[...] = h * dinv

    return pl.pallas_call(
        body,
        out_shape=jax.ShapeDtypeStruct((N, D), jnp.float32),
    )(acc, m, dinv, b, a, w_next)


def _tc_final(acc, m, dinv, b, seg, wl, bl):
    def body(acc_ref, m_ref, dinv_ref, b_ref, seg_ref, wl_ref, bl_ref, out_ref):
        h = _combine(acc_ref, m_ref) * dinv_ref[...] + b_ref[...]
        gid = lax.broadcasted_iota(jnp.int32, (N, G), 1)
        oh = (seg_ref[...] == gid).astype(jnp.float32)
        dn = (((0,), (0,)), ((), ()))
        sums = lax.dot_general(oh, h, dn, preferred_element_type=jnp.float32)
        cnt = lax.dot_general(oh, jnp.ones((N, 1), jnp.float32), dn,
                              preferred_element_type=jnp.float32)
        pooled = sums / jnp.maximum(cnt, 1.0)
        out = jnp.dot(pooled, wl_ref[...], preferred_element_type=jnp.float32)
        out_ref[...] = out + bl_ref[...]

    return pl.pallas_call(
        body,
        out_shape=jax.ShapeDtypeStruct((G, D), jnp.float32),
    )(acc, m, dinv, b, seg, wl, bl)


def kernel(x, edge_index, batch, W1, b1, a1, W2, b2, a2, W3, b3, Wl, bl):
    src = edge_index[0].astype(jnp.int32)
    dst = edge_index[1].astype(jnp.int32)
    pad = EPAD - E
    src = jnp.concatenate([src, jnp.zeros((pad,), jnp.int32)])
    dst = jnp.concatenate([dst, jnp.full((pad,), N, jnp.int32)])
    dsts = dst.reshape(TOT_CHUNKS, 1, CH)

    in0 = dst < NHALF
    in1 = (dst >= NHALF) & (dst < N)
    srcs2 = jnp.stack([jnp.where(in0, src, IGN), jnp.where(in1, src, IGN)])
    dsts2 = jnp.stack([jnp.where(in0, dst, IGN), jnp.where(in1, dst - NHALF, IGN)])
    srcs2 = srcs2.reshape(NC, TOT_CHUNKS, 1, CH)
    dsts2 = dsts2.reshape(NC, TOT_CHUNKS, 1, CH)

    deg_parts = _sc_degree(dsts)
    deg_t = deg_parts[:, :N].T

    m1, dinv = _tc_first(x, W1, deg_t)
    acc1 = _sc_aggregate(m1, srcs2, dsts2)
    m2 = _tc_mid(acc1, m1, dinv, b1.reshape(1, D), a1.reshape(1, 1), W2)
    acc2 = _sc_aggregate(m2, srcs2, dsts2)
    m3 = _tc_mid(acc2, m2, dinv, b2.reshape(1, D), a2.reshape(1, 1), W3)
    acc3 = _sc_aggregate(m3, srcs2, dsts2)

    return _tc_final(acc3, m3, dinv, b3.reshape(1, D),
                     batch.reshape(N, 1).astype(jnp.int32), Wl, bl)


# speedup vs baseline: 1.2470x; 1.2470x over previous
import functools

import jax
import jax.numpy as jnp
from jax import lax
from jax.experimental import pallas as pl
from jax.experimental.pallas import tpu as pltpu
from jax.experimental.pallas import tpu_sc as plsc

N = 10000
D = 128
E = 320000
G = 64

NC = 2
NS = 16
NW = NC * NS

CH = 128
TOT_CHUNKS = NW * (-(-E // (NW * CH)))
EPAD = TOT_CHUNKS * CH
CPW_DEG = TOT_CHUNKS // NW
CPW = TOT_CHUNKS // NS

NHIST = 10240
NHALF = N // 2
NACC = 5120
RPT = NACC // NS
ZCH = 32
NBUF = 2
IGN = -1


def _sc_mesh():
    return plsc.VectorSubcoreMesh(
        core_axis_name="c", subcore_axis_name="s", num_cores=NC, num_subcores=NS
    )


def _sc_degree(dsts):
    @functools.partial(
        pl.kernel,
        out_type=jax.ShapeDtypeStruct((NW, NHIST), jnp.float32),
        mesh=_sc_mesh(),
        compiler_params=pltpu.CompilerParams(needs_layout_passes=False),
        scratch_types=[
            pltpu.VMEM((CPW_DEG, 1, CH), jnp.int32),
            pltpu.VMEM((NHIST,), jnp.float32),
        ],
    )
    def deg_kernel(dst_hbm, out_hbm, dbuf, hist):
        c = lax.axis_index("c")
        s = lax.axis_index("s")
        wid = c * NS + s
        base = wid * CPW_DEG
        pltpu.sync_copy(dst_hbm.at[pl.ds(base, CPW_DEG)], dbuf)
        zero16 = jnp.zeros((16,), jnp.float32)

        @pl.loop(0, NHIST // 16)
        def _zero(i):
            hist[pl.ds(i * 16, 16)] = zero16

        ones16 = jnp.ones((16,), jnp.float32)

        @pl.loop(0, CPW_DEG)
        def _edges(j):
            for v in range(CH // 16):
                idx = dbuf[j, 0, pl.ds(v * 16, 16)]
                plsc.addupdate_scatter(hist, [idx], ones16)

        pltpu.sync_copy(hist, out_hbm.at[wid])

    return deg_kernel(dsts)


def _sc_aggregate(m, srcs2, dsts2):
    @functools.partial(
        pl.kernel,
        out_type=jax.ShapeDtypeStruct((NC, NACC, D), jnp.float32),
        mesh=_sc_mesh(),
        scratch_types=[
            pltpu.VMEM((CPW, 1, CH), jnp.int32),
            pltpu.VMEM((CPW, 1, CH), jnp.int32),
            pltpu.VMEM((NBUF, CH, D), jnp.float32),
            pltpu.VMEM((ZCH, D), jnp.float32),
            pltpu.VMEM_SHARED((NACC, D), jnp.float32),
            pltpu.SemaphoreType.DMA,
        ],
    )
    def agg_kernel(m_hbm, src_hbm, dst_hbm, out_hbm, sidx, didx, rows, zbuf,
                   acc, gsem):
        c = lax.axis_index("c")
        s = lax.axis_index("s")
        base = s * CPW
        pltpu.sync_copy(src_hbm.at[c, pl.ds(base, CPW)], sidx)
        pltpu.sync_copy(dst_hbm.at[c, pl.ds(base, CPW)], didx)
        zero16 = jnp.zeros((16,), jnp.float32)

        @pl.loop(0, ZCH)
        def _zrow(i):
            for v in range(D // 16):
                zbuf[i, pl.ds(v * 16, 16)] = zero16

        rbase = s * RPT

        @pl.loop(0, RPT // ZCH)
        def _zacc(i):
            pltpu.sync_copy(zbuf, acc.at[pl.ds(rbase + i * ZCH, ZCH)])

        plsc.subcore_barrier()

        def _gather(j, b):
            pltpu.async_copy(
                m_hbm.at[plsc.Indices(sidx.at[j, 0], ignored_value=IGN)],
                rows.at[b], gsem)

        for j in range(NBUF):
            _gather(j, j)

        @pl.loop(0, CPW)
        def _edges(j):
            b = lax.rem(j, NBUF)
            pltpu.make_async_copy(
                m_hbm.at[plsc.Indices(sidx.at[j, 0], ignored_value=IGN)],
                rows.at[b], gsem).wait()
            pltpu.sync_copy(
                rows.at[b],
                acc.at[plsc.Indices(didx.at[j, 0], ignored_value=IGN)],
                add=True)

            @pl.when(j + NBUF < CPW)
            def _refill():
                _gather(j + NBUF, b)

        plsc.subcore_barrier()
        pltpu.sync_copy(acc.at[pl.ds(rbase, RPT)], out_hbm.at[c, pl.ds(rbase, RPT)])

    return agg_kernel(m, srcs2, dsts2)


def _tc_first(x, W1, deg_t):
    def body(x_ref, w_ref, deg_ref, m_ref, dinv_ref):
        deg = jnp.sum(deg_ref[...], axis=1, keepdims=True) + 1.0
        dinv = lax.rsqrt(jnp.maximum(deg, 1.0))
        dinv_ref[...] = dinv
        h = jnp.dot(x_ref[...], w_ref[...], preferred_element_type=jnp.float32)
        m_ref[...] = h * dinv

    return pl.pallas_call(
        body,
        out_shape=[
            jax.ShapeDtypeStruct((N, D), jnp.float32),
            jax.ShapeDtypeStruct((N, 1), jnp.float32),
        ],
    )(x, W1, deg_t)


def _combine(acc_ref, m_ref):
    agg = jnp.concatenate(
        [acc_ref[0, pl.ds(0, NHALF)], acc_ref[1, pl.ds(0, NHALF)]], axis=0)
    return agg + m_ref[...]


def _tc_mid(acc, m, dinv, b, a, w_next):
    def body(acc_ref, m_ref, dinv_ref, b_ref, a_ref, w_ref, out_ref):
        dinv = dinv_ref[...]
        t = _combine(acc_ref, m_ref) * dinv + b_ref[...]
        z = jnp.where(t >= 0, t, a_ref[0, 0] * t)
        h = jnp.dot(z, w_ref[...], preferred_element_type=jnp.float32)
        out_ref[...] = h * dinv

    return pl.pallas_call(
        body,
        out_shape=jax.ShapeDtypeStruct((N, D), jnp.float32),
    )(acc, m, dinv, b, a, w_next)


def _tc_final(acc, m, dinv, b, seg, wl, bl):
    def body(acc_ref, m_ref, dinv_ref, b_ref, seg_ref, wl_ref, bl_ref, out_ref):
        h = _combine(acc_ref, m_ref) * dinv_ref[...] + b_ref[...]
        gid = lax.broadcasted_iota(jnp.int32, (N, G), 1)
        oh = (seg_ref[...] == gid).astype(jnp.float32)
        dn = (((0,), (0,)), ((), ()))
        sums = lax.dot_general(oh, h, dn, preferred_element_type=jnp.float32)
        cnt = lax.dot_general(oh, jnp.ones((N, 1), jnp.float32), dn,
                              preferred_element_type=jnp.float32)
        pooled = sums / jnp.maximum(cnt, 1.0)
        out = jnp.dot(pooled, wl_ref[...], preferred_element_type=jnp.float32)
        out_ref[...] = out + bl_ref[...]

    return pl.pallas_call(
        body,
        out_shape=jax.ShapeDtypeStruct((G, D), jnp.float32),
    )(acc, m, dinv, b, seg, wl, bl)


def kernel(x, edge_index, batch, W1, b1, a1, W2, b2, a2, W3, b3, Wl, bl):
    src = edge_index[0].astype(jnp.int32)
    dst = edge_index[1].astype(jnp.int32)
    pad = EPAD - E
    src = jnp.concatenate([src, jnp.zeros((pad,), jnp.int32)])
    dst = jnp.concatenate([dst, jnp.full((pad,), N, jnp.int32)])
    dsts = dst.reshape(TOT_CHUNKS, 1, CH)

    in0 = dst < NHALF
    in1 = (dst >= NHALF) & (dst < N)
    srcs2 = jnp.stack([jnp.where(in0, src, IGN), jnp.where(in1, src, IGN)])
    dsts2 = jnp.stack([jnp.where(in0, dst, IGN), jnp.where(in1, dst - NHALF, IGN)])
    srcs2 = srcs2.reshape(NC, TOT_CHUNKS, 1, CH)
    dsts2 = dsts2.reshape(NC, TOT_CHUNKS, 1, CH)

    deg_parts = _sc_degree(dsts)
    deg_t = deg_parts[:, :N].T

    m1, dinv = _tc_first(x, W1, deg_t)
    acc1 = _sc_aggregate(m1, srcs2, dsts2)
    m2 = _tc_mid(acc1, m1, dinv, b1.reshape(1, D), a1.reshape(1, 1), W2)
    acc2 = _sc_aggregate(m2, srcs2, dsts2)
    m3 = _tc_mid(acc2, m2, dinv, b2.reshape(1, D), a2.reshape(1, 1), W3)
    acc3 = _sc_aggregate(m3, srcs2, dsts2)

    return _tc_final(acc3, m3, dinv, b3.reshape(1, D),
                     batch.reshape(N, 1).astype(jnp.int32), Wl, bl)
